# baseline (speedup 1.0000x reference)
import functools

import jax
import jax.numpy as jnp
from jax.experimental import pallas as pl
from jax.experimental.pallas import tpu as pltpu

_SIGMA = (1.0, 1.0, 1.0)
_T = 8
_K = 4


def _gauss_matrix(n, scale):
    i = jax.lax.broadcasted_iota(jnp.int32, (n, n), 0)
    j = jax.lax.broadcasted_iota(jnp.int32, (n, n), 1)
    d = (i - j).astype(jnp.float32) * scale
    k = jnp.exp(-0.5 * d * d)
    return k / jnp.sum(k, axis=1, keepdims=True)


def _kron_gauss_eye(d, scale):
    n = d * _T
    a = jax.lax.broadcasted_iota(jnp.int32, (n, n), 0)
    b = jax.lax.broadcasted_iota(jnp.int32, (n, n), 1)
    dz = ((a // _T) - (b // _T)).astype(jnp.float32) * scale
    k = jnp.exp(-0.5 * dz * dz)
    k = jnp.where((a % _T) == (b % _T), k, 0.0)
    return k / jnp.sum(k, axis=1, keepdims=True)


def _fused_kernel(v_ref, x_ref, o_ref, p_ref, ay_ref, ax_ref, azk_ref,
                  *, nchan):
    dq, h, w = x_ref.shape[1], x_ref.shape[2], x_ref.shape[3]
    d = p_ref.shape[0] // 2
    hq = h // _K
    ci = pl.program_id(0)
    k = pl.program_id(1)

    @pl.when(jnp.logical_and(ci == 0, k == 0))
    def _init():
        ay_ref[...] = _gauss_matrix(h, v_ref[2] / _SIGMA[2])
        ax_ref[...] = _gauss_matrix(w, v_ref[1] / _SIGMA[1])
        azk_ref[...] = _kron_gauss_eye(d, v_ref[0] / _SIGMA[0])

    @pl.when(ci < nchan)
    def _xy():
        t = jax.lax.dot_general(
            x_ref[0].reshape(dq * h, w), ax_ref[...],
            (((1,), (1,)), ((), ())),
            preferred_element_type=jnp.float32).reshape(dq, h, w)
        base = (ci % 2) * d + k * dq
        for i in range(dq):
            p_ref[pl.ds(base + i, 1)] = jnp.dot(
                ay_ref[...], t[i],
                preferred_element_type=jnp.float32)[None]

    @pl.when(ci > 0)
    def _z():
        zbase = ((ci - 1) % 2) * d
        for hb in range(hq // _T):
            blk = p_ref[pl.ds(zbase, d), pl.ds(k * hq + hb * _T, _T), :]
            ob = jnp.dot(azk_ref[...], blk.reshape(d * _T, w),
                         preferred_element_type=jnp.float32)
            o_ref[0, :, hb * _T:(hb + 1) * _T, :] = ob.reshape(d, _T, w)


@jax.jit
def kernel(input_, image, v_gamma):
    c, d, h, w = input_.shape
    dq = d // _K
    body = functools.partial(_fused_kernel, nchan=c)
    return pl.pallas_call(
        body,
        grid=(c + 1, _K),
        in_specs=[
            pl.BlockSpec(memory_space=pltpu.SMEM),
            pl.BlockSpec((1, dq, h, w),
                         lambda ci, k: (jnp.minimum(ci, c - 1),
                                        jnp.where(ci < c, k, _K - 1), 0, 0)),
        ],
        out_specs=pl.BlockSpec((1, d, h // _K, w),
                               lambda ci, k: (jnp.maximum(ci - 1, 0), 0,
                                              jnp.where(ci < 1, 0, k), 0)),
        out_shape=jax.ShapeDtypeStruct((c, d, h, w), jnp.float32),
        scratch_shapes=[
            pltpu.VMEM((2 * d, h, w), jnp.float32),
            pltpu.VMEM((h, h), jnp.float32),
            pltpu.VMEM((w, w), jnp.float32),
            pltpu.VMEM((d * _T, d * _T), jnp.float32),
        ],
        compiler_params=pltpu.CompilerParams(
            dimension_semantics=("arbitrary", "arbitrary")),
    )(v_gamma, input_)


# speedup vs baseline: 2.0458x; 2.0458x over previous
import jax
import jax.numpy as jnp
from jax.experimental import pallas as pl
from jax.experimental.pallas import tpu as pltpu

_SIGMA = (1.0, 1.0, 1.0)
_T = 8
_CB = 2


def _gauss_matrix(n, scale):
    i = jax.lax.broadcasted_iota(jnp.int32, (n, n), 0)
    j = jax.lax.broadcasted_iota(jnp.int32, (n, n), 1)
    d = (i - j).astype(jnp.float32) * scale
    k = jnp.exp(-0.5 * d * d)
    return k / jnp.sum(k, axis=1, keepdims=True)


def _kron_gauss_eye(d, scale):
    n = d * _T
    a = jax.lax.broadcasted_iota(jnp.int32, (n, n), 0)
    b = jax.lax.broadcasted_iota(jnp.int32, (n, n), 1)
    dz = ((a // _T) - (b // _T)).astype(jnp.float32) * scale
    k = jnp.exp(-0.5 * dz * dz)
    k = jnp.where((a % _T) == (b % _T), k, 0.0)
    return k / jnp.sum(k, axis=1, keepdims=True)


def _fused_kernel(v_ref, x_ref, o_ref, p_ref):
    cb, d, h, w = x_ref.shape
    ay = _gauss_matrix(h, v_ref[2] / _SIGMA[2])
    ax = _gauss_matrix(w, v_ref[1] / _SIGMA[1])
    azk = _kron_gauss_eye(d, v_ref[0] / _SIGMA[0])

    t = jax.lax.dot_general(
        x_ref[...].reshape(cb * d * h, w), ax, (((1,), (1,)), ((), ())),
        preferred_element_type=jnp.float32).reshape(cb * d, h, w)
    for di in range(cb * d):
        p_ref[di] = jnp.dot(ay, t[di], preferred_element_type=jnp.float32)

    for cj in range(cb):
        for hb in range(h // _T):
            blk = p_ref[cj * d:(cj + 1) * d, hb * _T:(hb + 1) * _T, :]
            ob = jnp.dot(azk, blk.reshape(d * _T, w),
                         preferred_element_type=jnp.float32)
            o_ref[cj, :, hb * _T:(hb + 1) * _T, :] = ob.reshape(d, _T, w)


@jax.jit
def kernel(input_, image, v_gamma):
    c, d, h, w = input_.shape
    return pl.pallas_call(
        _fused_kernel,
        grid=(c // _CB,),
        in_specs=[
            pl.BlockSpec(memory_space=pltpu.SMEM),
            pl.BlockSpec((_CB, d, h, w), lambda ci: (ci, 0, 0, 0)),
        ],
        out_specs=pl.BlockSpec((_CB, d, h, w), lambda ci: (ci, 0, 0, 0)),
        out_shape=jax.ShapeDtypeStruct((c, d, h, w), jnp.float32),
        scratch_shapes=[pltpu.VMEM((_CB * d, h, w), jnp.float32)],
        compiler_params=pltpu.CompilerParams(
            dimension_semantics=("arbitrary",)),
    )(v_gamma, input_)
